# Initial kernel scaffold; baseline (speedup 1.0000x reference)
import jax
import jax.numpy as jnp
from jax.experimental import pallas as pl


def kernel(x1, x2, params):
    raise NotImplementedError("write your pallas kernel here")


# speedup vs baseline: 1.6459x; 1.6459x over previous
import jax
import jax.numpy as jnp
from jax import lax
from jax.experimental import pallas as pl
from jax.experimental.pallas import tpu as pltpu

_N = 20000
_BLK = 1000
_GRID = _N // _BLK
_NEG = -1e30


def _fold_body(wp_ref, wvf_ref, bp_ref, bvf_ref, wpf_ref, bpf_ref):
    wpf_ref[...] = jnp.dot(wp_ref[...], wvf_ref[...],
                           preferred_element_type=jnp.float32)
    bpf_ref[...] = jnp.dot(bp_ref[...], wvf_ref[...],
                           preferred_element_type=jnp.float32) + bvf_ref[...]


def _row_scale(vec12, nrows, ncols):
    rows = lax.broadcasted_iota(jnp.int32, (nrows, ncols), 0)
    return jnp.where(rows == 0, vec12[0:1, 0:1], vec12[0:1, 1:2])


def _pool_body(x1_ref, x2_ref, wpf_ref, bpf_ref,
               wva_ref, bva_ref, wvb_ref, bvb_ref, wvc_ref, bvc_ref,
               wuf_ref, buf_ref,
               wua_ref, bua_ref, wub_ref, bub_ref, wuc_ref, buc_ref,
               m1_ref, m2_ref,
               mv_ref, sv_ref, mu_ref, su_ref):
    i = pl.program_id(0)

    @pl.when(i == 0)
    def _init():
        mv_ref[...] = jnp.full((1, 2), _NEG, jnp.float32)
        mu_ref[...] = jnp.full((1, 2), _NEG, jnp.float32)
        sv_ref[...] = jnp.zeros((1, 2), jnp.float32)
        su_ref[...] = jnp.zeros((1, 2), jnp.float32)
        m1_ref[...] = jnp.zeros((2, 512), jnp.float32)
        m2_ref[...] = jnp.zeros((2, 512), jnp.float32)

    def branch(h, wa, ba, wb, bb, wc, bc, m_ref, s_ref, v_ref):
        ga = jnp.tanh(jnp.dot(h, wa, preferred_element_type=jnp.float32) + ba)
        gb = jax.nn.sigmoid(jnp.dot(h, wb, preferred_element_type=jnp.float32) + bb)
        l = jnp.dot(ga * gb, wc, preferred_element_type=jnp.float32) + bc
        bm = jnp.max(l, axis=0, keepdims=True)
        nm = jnp.maximum(m_ref[...], bm)
        sc = jnp.exp(m_ref[...] - nm)
        p = jnp.exp(l - nm)
        s_ref[...] = s_ref[...] * sc + jnp.sum(p, axis=0, keepdims=True)
        m_ref[...] = nm
        pv = lax.dot_general(p, h, (((0,), (0,)), ((), ())),
                             preferred_element_type=jnp.float32)
        v_ref[...] = v_ref[...] * _row_scale(sc, 2, 512) + pv

    hv = jnp.maximum(
        jnp.dot(x1_ref[...], wpf_ref[...], preferred_element_type=jnp.float32)
        + bpf_ref[...], 0.0)
    branch(hv, wva_ref[...], bva_ref[...], wvb_ref[...], bvb_ref[...],
           wvc_ref[...], bvc_ref[...], mv_ref, sv_ref, m1_ref)

    hu = jnp.maximum(
        jnp.dot(x2_ref[...], wuf_ref[...], preferred_element_type=jnp.float32)
        + buf_ref[...], 0.0)
    branch(hu, wua_ref[...], bua_ref[...], wub_ref[...], bub_ref[...],
           wuc_ref[...], buc_ref[...], mu_ref, su_ref, m2_ref)

    @pl.when(i == _GRID - 1)
    def _norm():
        m1_ref[...] = m1_ref[...] * _row_scale(1.0 / sv_ref[...], 2, 512)
        m2_ref[...] = m2_ref[...] * _row_scale(1.0 / su_ref[...], 2, 512)


def _head_body(m1_ref, m2_ref,
               w1a_ref, b1a_ref, w1b_ref, b1b_ref,
               w3a_ref, b3a_ref, w3b_ref, b3b_ref,
               w2a_ref, b2a_ref, w2b_ref, b2b_ref, wop_ref, bop_ref,
               wg1_ref, bg1_ref, wg2_ref, bg2_ref, wc_ref, bc_ref,
               out_ref):
    m1 = m1_ref[...]
    m2 = m2_ref[...]
    cat = jnp.concatenate([m1, m2], axis=1)

    def mm(a, w, b):
        return jnp.dot(a, w, preferred_element_type=jnp.float32) + b

    e1 = jnp.maximum(mm(jnp.maximum(mm(m1, w1a_ref[...], b1a_ref[...]), 0.0),
                        w1b_ref[...], b1b_ref[...]), 0.0)
    e3 = jnp.maximum(mm(jnp.maximum(mm(m2, w3a_ref[...], b3a_ref[...]), 0.0),
                        w3b_ref[...], b3b_ref[...]), 0.0)
    e2 = mm(jnp.maximum(mm(jnp.maximum(mm(cat, w2a_ref[...], b2a_ref[...]), 0.0),
                           w2b_ref[...], b2b_ref[...]), 0.0),
            wop_ref[...], bop_ref[...])

    z = mm(jnp.maximum(mm(cat, wg1_ref[...], bg1_ref[...]), 0.0),
           wg2_ref[...], bg2_ref[...])
    z = z - jnp.max(z, axis=1, keepdims=True)
    ez = jnp.exp(z)
    g = ez / jnp.sum(ez, axis=1, keepdims=True)

    fused = (g[:, 0:1] * e1 + g[:, 1:2] * e2 + g[:, 2:3] * e3)
    out_ref[...] = jnp.sum(fused * wc_ref[...], axis=1, keepdims=True) + bc_ref[...]


def _full(shape):
    return pl.BlockSpec(shape, lambda i: tuple(0 for _ in shape))


def kernel(x1, x2, params):
    (Wp, bp, Wvf, bvf, Wva, bva, Wvb, bvb, Wvc, bvc,
     Wuf, buf, Wua, bua, Wub, bub, Wuc, buc,
     W1a, b1a, W1b, b1b, W3a, b3a, W3b, b3b,
     W2a, b2a, W2b, b2b, Wop, bop,
     Wg1, bg1, Wg2, bg2, Wc, bc) = params

    r = lambda b: b.reshape(1, -1)

    wpf, bpf = pl.pallas_call(
        _fold_body,
        out_shape=[jax.ShapeDtypeStruct((2560, 512), jnp.float32),
                   jax.ShapeDtypeStruct((1, 512), jnp.float32)],
    )(Wp, Wvf, r(bp), r(bvf))

    m1, m2 = pl.pallas_call(
        _pool_body,
        grid=(_GRID,),
        in_specs=[
            pl.BlockSpec((_BLK, 2560), lambda i: (i, 0)),
            pl.BlockSpec((_BLK, 1024), lambda i: (i, 0)),
            _full((2560, 512)), _full((1, 512)),
            _full((512, 256)), _full((1, 256)),
            _full((512, 256)), _full((1, 256)),
            _full((256, 2)), _full((1, 2)),
            _full((1024, 512)), _full((1, 512)),
            _full((512, 256)), _full((1, 256)),
            _full((512, 256)), _full((1, 256)),
            _full((256, 2)), _full((1, 2)),
        ],
        out_specs=[_full((2, 512)), _full((2, 512))],
        out_shape=[jax.ShapeDtypeStruct((2, 512), jnp.float32),
                   jax.ShapeDtypeStruct((2, 512), jnp.float32)],
        scratch_shapes=[pltpu.VMEM((1, 2), jnp.float32)] * 4,
        compiler_params=pltpu.CompilerParams(
            dimension_semantics=("arbitrary",)),
    )(x1, x2, wpf, bpf,
      Wva, r(bva), Wvb, r(bvb), Wvc, r(bvc),
      Wuf, r(buf),
      Wua, r(bua), Wub, r(bub), Wuc, r(buc))

    out = pl.pallas_call(
        _head_body,
        out_shape=jax.ShapeDtypeStruct((2, 1), jnp.float32),
    )(m1, m2,
      W1a, r(b1a), W1b, r(b1b),
      W3a, r(b3a), W3b, r(b3b),
      W2a, r(b2a), W2b, r(b2b), Wop, r(bop),
      Wg1, r(bg1), Wg2, r(bg2), Wc, bc.reshape(2, 1))

    return out.reshape(1, 2)
